# baseline (speedup 1.0000x reference)
import jax
import jax.numpy as jnp
from jax.experimental import pallas as pl
from jax.experimental.pallas import tpu as pltpu

_sig = jax.nn.sigmoid
_tanh = jnp.tanh

_LCH = 2048


def _level_ranges(n_full):
    levels = []
    l = 0
    while (2 ** l - 1) < n_full:
        lo = 2 ** l - 1
        hi = min(2 ** (l + 1) - 2, n_full - 1)
        levels.append((lo, hi))
        l += 1
    return list(reversed(levels))


def _tree_kernel(xl_ref, cl_ref, wi_ref, bi_ref, ulvl_ref, biou_ref, ufb_ref,
                 sw_ref, sb_ref, w_ref, out_ref,
                 ha, ca, hb, cb, hc, cc, iou_s, out_s):
    n = out_ref.shape[0]
    s = cl_ref.shape[1]
    n_full = (n - 1) // 2
    levels = _level_ranges(n_full)
    L0 = n_full + 1
    nl = n + 1 - L0
    leaf_chunks = [(r, min(r + _LCH, nl)) for r in range(0, nl, _LCH)]
    f32 = jnp.float32
    bf16 = jnp.bfloat16

    def accv(iou_val, k):
        return (iou_val[:, :s] * w_ref[3 * k]
                + iou_val[:, s:2 * s] * w_ref[3 * k + 1]
                + iou_val[:, 2 * s:] * w_ref[3 * k + 2])

    def leaf_gates(val, c_prev):
        ib = val + biou_ref[:]
        c_new = _sig(ib[:, :s]) * _tanh(ib[:, 2 * s:]) + c_prev
        h_new = _sig(ib[:, s:2 * s]) * _tanh(c_new)
        return h_new, c_new

    def init_fused(h_o, c_o):
        for (r, e) in leaf_chunks:
            val = (jnp.dot(xl_ref[r:e].astype(bf16), wi_ref[:],
                           preferred_element_type=f32) + bi_ref[:])
            iou_s[r:e] = val.astype(bf16)
            h_new, c_new = leaf_gates(val, cl_ref[r:e])
            h_o[r:e] = h_new
            c_o[r:e] = c_new
            out_s[L0 + r:L0 + e] = accv(val, 0) + w_ref[13]

    def pairmerge(v):
        return v.reshape(v.shape[0] // 2, 2 * v.shape[1])

    def levels_pass(acc_k, h_l, c_l):
        lo0, hi0 = levels[0]
        a0 = 2 * lo0 + 2 - L0
        b0 = 2 * hi0 + 4 - L0
        hcat = pairmerge(h_l[a0:b0])
        ccat = pairmerge(c_l[a0:b0])
        for i, (lo, hi) in enumerate(levels):
            m = hi - lo + 1
            z = jnp.dot(hcat.astype(bf16), ulvl_ref[:],
                        preferred_element_type=f32)
            fg = _sig(z[:, :2 * s] + ufb_ref[:])
            iou_n = z[:, 2 * s:]
            c_red = fg[:, :s] * ccat[:, :s] + fg[:, s:] * ccat[:, s:]
            ib = iou_n + biou_ref[:]
            c_new = _sig(ib[:, :s]) * _tanh(ib[:, 2 * s:]) + c_red
            h_new = _sig(ib[:, s:2 * s]) * _tanh(c_new)
            if acc_k is None:
                out_s[lo + 1:hi + 2] += h_new * w_ref[12]
            elif acc_k == 0:
                out_s[lo + 1:hi + 2] = accv(iou_n, 0) + w_ref[13]
            else:
                out_s[lo + 1:hi + 2] += accv(iou_n, acc_k)
            if i + 1 < len(levels):
                if i == 0:
                    hn = jnp.concatenate([h_new, h_l[0:a0]], axis=0)
                    cn = jnp.concatenate([c_new, c_l[0:a0]], axis=0)
                    hcat = pairmerge(hn)
                    ccat = pairmerge(cn)
                else:
                    hcat = pairmerge(h_new)
                    ccat = pairmerge(c_new)

    def stage_fused(ix, k, c_i, h_o, c_o):
        for (r, e) in leaf_chunks:
            val = jnp.maximum(
                jnp.dot(iou_s[r:e], sw_ref[ix],
                        preferred_element_type=f32) + sb_ref[ix:ix + 1, :],
                0.0)
            iou_s[r:e] = val.astype(bf16)
            h_new, c_new = leaf_gates(val, c_i[r:e])
            h_o[r:e] = h_new
            c_o[r:e] = c_new
            out_s[L0 + r:L0 + e] += accv(val, k)

    def final_leaf(c_i, h_o, c_o):
        for (r, e) in leaf_chunks:
            h_new, c_new = leaf_gates(iou_s[r:e].astype(f32), c_i[r:e])
            h_o[r:e] = h_new
            c_o[r:e] = c_new
            out_s[L0 + r:L0 + e] += h_new * w_ref[12]

    init_fused(ha, ca)
    stage_fused(0, 1, ca, hb, cb)
    levels_pass(0, ha, ca)
    stage_fused(1, 2, cb, hc, cc)
    levels_pass(1, hb, cb)
    stage_fused(2, 3, cc, ha, ca)
    levels_pass(2, hc, cc)
    final_leaf(ca, hb, cb)
    levels_pass(3, ha, ca)
    levels_pass(None, hb, cb)
    out_ref[:] = out_s[1:n + 1]


def kernel(x, h, c, W_init, b_init, U_iou_w, b_iou, U_f_w, U_f_b,
           stage_W, stage_b, out_w, out_b):
    n, s = c.shape
    n_full = (n - 1) // 2
    nl = ((n - n_full) + 7) // 8 * 8
    wvec = jnp.concatenate([out_w, out_b]).astype(jnp.float32)
    u_lvl = jnp.concatenate([U_f_w.T, U_iou_w.T], axis=1)
    leaf_f32 = pltpu.VMEM((nl, s), jnp.float32)
    out = pl.pallas_call(
        _tree_kernel,
        out_shape=jax.ShapeDtypeStruct((n, s), jnp.float32),
        in_specs=[pl.BlockSpec(memory_space=pltpu.VMEM)] * 9
        + [pl.BlockSpec(memory_space=pltpu.SMEM)],
        out_specs=pl.BlockSpec(memory_space=pltpu.VMEM),
        scratch_shapes=[
            leaf_f32, leaf_f32,
            leaf_f32, leaf_f32,
            leaf_f32, leaf_f32,
            pltpu.VMEM((nl, 3 * s), jnp.bfloat16),
            pltpu.VMEM((n + 8, s), jnp.float32),
        ],
        compiler_params=pltpu.CompilerParams(
            vmem_limit_bytes=120 * 1024 * 1024),
    )(x[n_full:], c[n_full:],
      W_init.T.astype(jnp.bfloat16), b_init.reshape(1, -1),
      u_lvl.astype(jnp.bfloat16), b_iou.reshape(1, -1),
      U_f_b.reshape(1, -1),
      jnp.transpose(stage_W, (0, 2, 1)).astype(jnp.bfloat16), stage_b,
      wvec)
    return out.reshape(n, 1, 1, s)


# speedup vs baseline: 1.0139x; 1.0139x over previous
import jax
import jax.numpy as jnp
from jax.experimental import pallas as pl
from jax.experimental.pallas import tpu as pltpu

_sig = jax.nn.sigmoid
_tanh = jnp.tanh

_LCH = 2048


def _level_ranges(n_full):
    levels = []
    l = 0
    while (2 ** l - 1) < n_full:
        lo = 2 ** l - 1
        hi = min(2 ** (l + 1) - 2, n_full - 1)
        levels.append((lo, hi))
        l += 1
    return list(reversed(levels))


def _tree_kernel(xl_ref, cl_ref, wi_ref, bi_ref, ulvl_ref, biou_ref, ufb_ref,
                 sw_ref, sb_ref, w_ref, out_ref, h_s, c_s, iou_s, out_s):
    n = out_ref.shape[0]
    s = cl_ref.shape[1]
    n_full = (n - 1) // 2
    levels = _level_ranges(n_full)
    L0 = n_full + 1
    leaf_chunks = [(r, min(r + _LCH, n + 1)) for r in range(L0, n + 1, _LCH)]
    f32 = jnp.float32
    bf16 = jnp.bfloat16

    def accv(iou_val, k):
        return (iou_val[:, :s] * w_ref[3 * k]
                + iou_val[:, s:2 * s] * w_ref[3 * k + 1]
                + iou_val[:, 2 * s:] * w_ref[3 * k + 2])

    def leaf_gates(val, c_prev):
        ib = val + biou_ref[:]
        c_new = _sig(ib[:, :s]) * _tanh(ib[:, 2 * s:]) + c_prev
        h_new = _sig(ib[:, s:2 * s]) * _tanh(c_new)
        return h_new, c_new

    for (r, e) in leaf_chunks:
        val = (jnp.dot(xl_ref[r - L0:e - L0].astype(bf16), wi_ref[:],
                       preferred_element_type=f32) + bi_ref[:])
        iou_s[r:e] = val.astype(bf16)
        h_new, c_new = leaf_gates(val, cl_ref[r - L0:e - L0])
        h_s[r:e] = h_new
        c_s[r:e] = c_new
        out_s[r:e] = accv(val, 0) + w_ref[13]

    def pairmerge(v):
        return v.reshape(v.shape[0] // 2, 2 * v.shape[1])

    def levels_pass(acc_k):
        lo0, hi0 = levels[0]
        hcat = pairmerge(h_s[2 * lo0 + 2:2 * hi0 + 4])
        ccat = pairmerge(c_s[2 * lo0 + 2:2 * hi0 + 4])
        for i, (lo, hi) in enumerate(levels):
            m = hi - lo + 1
            z = jnp.dot(hcat.astype(bf16), ulvl_ref[:],
                        preferred_element_type=f32)
            fg = _sig(z[:, :2 * s] + ufb_ref[:])
            iou_n = z[:, 2 * s:]
            c_red = fg[:, :s] * ccat[:, :s] + fg[:, s:] * ccat[:, s:]
            ib = iou_n + biou_ref[:]
            c_new = _sig(ib[:, :s]) * _tanh(ib[:, 2 * s:]) + c_red
            h_new = _sig(ib[:, s:2 * s]) * _tanh(c_new)
            if acc_k is None:
                out_s[lo + 1:hi + 2] += h_new * w_ref[12]
            elif acc_k == 0:
                out_s[lo + 1:hi + 2] = accv(iou_n, 0) + w_ref[13]
            else:
                out_s[lo + 1:hi + 2] += accv(iou_n, acc_k)
            if i + 1 < len(levels):
                if i == 0:
                    lo1, hi1 = levels[1]
                    hn = jnp.concatenate([h_new, h_s[L0:2 * hi1 + 4]], axis=0)
                    cn = jnp.concatenate([c_new, c_s[L0:2 * hi1 + 4]], axis=0)
                    hcat = pairmerge(hn)
                    ccat = pairmerge(cn)
                else:
                    hcat = pairmerge(h_new)
                    ccat = pairmerge(c_new)

    def stage_fused(ix, k):
        for (r, e) in leaf_chunks:
            val = jnp.maximum(
                jnp.dot(iou_s[r:e], sw_ref[ix],
                        preferred_element_type=f32) + sb_ref[ix:ix + 1, :],
                0.0)
            iou_s[r:e] = val.astype(bf16)
            h_new, c_new = leaf_gates(val, c_s[r:e])
            h_s[r:e] = h_new
            c_s[r:e] = c_new
            out_s[r:e] += accv(val, k)

    levels_pass(0)
    for ix in range(3):
        stage_fused(ix, ix + 1)
        levels_pass(ix + 1)
    for (r, e) in leaf_chunks:
        h_new, c_new = leaf_gates(iou_s[r:e].astype(f32), c_s[r:e])
        h_s[r:e] = h_new
        c_s[r:e] = c_new
        out_s[r:e] += h_new * w_ref[12]
    levels_pass(None)
    out_ref[:] = out_s[1:n + 1]


def kernel(x, h, c, W_init, b_init, U_iou_w, b_iou, U_f_w, U_f_b,
           stage_W, stage_b, out_w, out_b):
    n, s = c.shape
    np_ = n + 8
    n_full = (n - 1) // 2
    wvec = jnp.concatenate([out_w, out_b]).astype(jnp.float32)
    u_lvl = jnp.concatenate([U_f_w.T, U_iou_w.T], axis=1)
    out = pl.pallas_call(
        _tree_kernel,
        out_shape=jax.ShapeDtypeStruct((n, s), jnp.float32),
        in_specs=[pl.BlockSpec(memory_space=pltpu.VMEM)] * 9
        + [pl.BlockSpec(memory_space=pltpu.SMEM)],
        out_specs=pl.BlockSpec(memory_space=pltpu.VMEM),
        scratch_shapes=[
            pltpu.VMEM((np_, s), jnp.float32),
            pltpu.VMEM((np_, s), jnp.float32),
            pltpu.VMEM((np_, 3 * s), jnp.bfloat16),
            pltpu.VMEM((np_, s), jnp.float32),
        ],
        compiler_params=pltpu.CompilerParams(
            vmem_limit_bytes=120 * 1024 * 1024),
    )(x[n_full:], c[n_full:],
      W_init.T.astype(jnp.bfloat16), b_init.reshape(1, -1),
      u_lvl.astype(jnp.bfloat16), b_iou.reshape(1, -1),
      U_f_b.reshape(1, -1),
      jnp.transpose(stage_W, (0, 2, 1)).astype(jnp.bfloat16), stage_b,
      wvec)
    return out.reshape(n, 1, 1, s)
